# Initial kernel scaffold; baseline (speedup 1.0000x reference)
import jax
import jax.numpy as jnp
from jax.experimental import pallas as pl


def kernel(x, edge_index, batch, W1, att_src1, att_dst1, b1, W2, att_src2, att_dst2, b2, Wl, bl):
    raise NotImplementedError("write your pallas kernel here")


# speedup vs baseline: 1.0434x; 1.0434x over previous
import functools

import jax
import jax.numpy as jnp
from jax.experimental import pallas as pl
from jax.experimental.pallas import tpu as pltpu


def _mm_kernel(x_ref, w_ref, o_ref):
    o_ref[...] = jnp.dot(x_ref[...], w_ref[...],
                         preferred_element_type=jnp.float32)


def _matmul(x, w, blk_m=400):
    m, k = x.shape
    k2, n = w.shape
    grid = (m // blk_m,)
    return pl.pallas_call(
        _mm_kernel,
        grid=grid,
        in_specs=[
            pl.BlockSpec((blk_m, k), lambda i: (i, 0)),
            pl.BlockSpec((k, n), lambda i: (0, 0)),
        ],
        out_specs=pl.BlockSpec((blk_m, n), lambda i: (i, 0)),
        out_shape=jax.ShapeDtypeStruct((m, n), jnp.float32),
    )(x, w)


def _gat_conv(x, src, dst, W, att_src, att_dst, bias):
    n = x.shape[0]
    heads, oc = att_src.shape
    h = _matmul(x, W).reshape(n, heads, oc)
    a_src = (h * att_src[None, :, :]).sum(-1)
    a_dst = (h * att_dst[None, :, :]).sum(-1)
    e = a_src[src] + a_dst[dst]
    e = jax.nn.leaky_relu(e, 0.2)
    p = jnp.exp(e)
    s = jax.ops.segment_sum(p, dst, num_segments=n)
    alpha = p / (s[dst] + 1e-16)
    msg = h[src] * alpha[..., None]
    out = jax.ops.segment_sum(msg, dst, num_segments=n)
    return out.mean(axis=1) + bias


def kernel(x, edge_index, batch, W1, att_src1, att_dst1, b1,
           W2, att_src2, att_dst2, b2, Wl, bl):
    n = x.shape[0]
    loops = jnp.arange(n, dtype=edge_index.dtype)
    src = jnp.concatenate([edge_index[0], loops])
    dst = jnp.concatenate([edge_index[1], loops])
    h = jax.nn.relu(_gat_conv(x, src, dst, W1, att_src1, att_dst1, b1))
    h = _gat_conv(h, src, dst, W2, att_src2, att_dst2, b2)
    counts = jax.ops.segment_sum(jnp.ones((n,), h.dtype), batch,
                                 num_segments=64)
    pooled = jax.ops.segment_sum(h, batch, num_segments=64)
    pooled = pooled / jnp.clip(counts, 1.0)[:, None]
    return pooled @ Wl + bl


# speedup vs baseline: 6.1369x; 5.8815x over previous
import functools

import jax
import jax.numpy as jnp
from jax import lax
from jax.experimental import pallas as pl
from jax.experimental.pallas import tpu as pltpu
from jax.experimental.pallas import tpu_sc as plsc

_N = 10000
_NP = 10240
_E_TOT = 330000
_EP = 335872
_EPT = _EP // 32
_CB = 128
_NCH = _EPT // _CB

_EPT2 = _EP // 16
_CC = 32
_NCHC = _EPT2 // _CC

_MESH = plsc.VectorSubcoreMesh(
    core_axis_name="c", subcore_axis_name="s", num_cores=2, num_subcores=16)


def _mm_kernel(x_ref, w_ref, o_ref):
    o_ref[...] = jnp.dot(x_ref[...], w_ref[...],
                         preferred_element_type=jnp.float32)


def _matmul(x, w, blk_m=400):
    m, k = x.shape
    _, n = w.shape
    return pl.pallas_call(
        _mm_kernel,
        grid=(m // blk_m,),
        in_specs=[
            pl.BlockSpec((blk_m, k), lambda i: (i, 0)),
            pl.BlockSpec((k, n), lambda i: (0, 0)),
        ],
        out_specs=pl.BlockSpec((blk_m, n), lambda i: (i, 0)),
        out_shape=jax.ShapeDtypeStruct((m, n), jnp.float32),
    )(x, w)


def _attn_body(asd, dsa, src, dst, z16, p_out, s_out,
               sidx, didx, srows, drows, pbuf, pflat, zbuf, zidx, sacc, sem):
    c = lax.axis_index("c")
    s = lax.axis_index("s")
    wid = s * 2 + c
    stripe = _NP // 16

    def zrow(j, cc):
        zbuf[j] = jnp.zeros((16,), jnp.float32)
        return cc

    lax.fori_loop(0, _CB, zrow, 0)

    def fill_zidx(off):
        iot = lax.iota(jnp.int32, 16)
        for j in range(_CB // 16):
            zidx[pl.ds(j * 16, 16)] = iot + (off + j * 16)

    def zcp(i, cc):
        fill_zidx(s * stripe + i * _CB)
        pltpu.sync_copy(zbuf, sacc.at[zidx])
        return cc

    lax.fori_loop(0, stripe // _CB, zcp, 0)
    plsc.subcore_barrier()
    base = wid * _EPT

    def chunk(k, carry):
        off = base + k * _CB
        pltpu.sync_copy(src.at[pl.ds(off, _CB)], sidx)
        pltpu.sync_copy(dst.at[pl.ds(off, _CB)], didx)
        pltpu.async_copy(asd.at[sidx], srows, sem).wait()
        pltpu.async_copy(dsa.at[didx], drows, sem).wait()

        def edge(j, cc):
            e = srows[j, pl.ds(0, 16)] + drows[j, pl.ds(0, 16)]
            e = jnp.where(e > 0, e, 0.2 * e)
            p = jnp.exp(e)
            pbuf[j] = p
            pflat[pl.ds(j * 16, 16)] = p
            return cc

        lax.fori_loop(0, _CB, edge, 0)
        pltpu.sync_copy(pflat, p_out.at[pl.ds(off * 16, _CB * 16)])
        pltpu.sync_copy(pbuf, sacc.at[didx], add=True)
        return carry

    lax.fori_loop(0, _NCH, chunk, 0)
    plsc.subcore_barrier()

    def fcp(i, cc):
        off = s * stripe + i * _CB
        fill_zidx(off)
        pltpu.sync_copy(sacc.at[zidx], zbuf)
        pltpu.sync_copy(zbuf, s_out.at[c, pl.ds(off, _CB)])
        return cc

    lax.fori_loop(0, stripe // _CB, fcp, 0)


def _make_msg(half):

    def _msg_body(ht, stab, pfl, src, dst, out,
                  sidx, didx, sadj, hrows, srows, pbuf, msg, zbuf, zidx,
                  sacc, sem):
        c = lax.axis_index("c")
        s = lax.axis_index("s")
        stripe = _NP // 16
        zero16 = jnp.zeros((16,), jnp.float32)

        def zrow(j, cc):
            for v in range(4):
                zbuf[j, pl.ds(v * 16, 16)] = zero16
            return cc

        lax.fori_loop(0, 128, zrow, 0)

        def fill_zidx(off):
            iot = lax.iota(jnp.int32, 16)
            for j in range(8):
                zidx[pl.ds(j * 16, 16)] = iot + (off + j * 16)

        def zcp(i, cc):
            fill_zidx(s * stripe + i * 128)
            pltpu.sync_copy(zbuf, sacc.at[zidx])
            return cc

        lax.fori_loop(0, stripe // 128, zcp, 0)
        plsc.subcore_barrier()
        base = s * _EPT2
        cbase = (c * 2 + half) * _NP

        def chunk(k, carry):
            off = base + k * _CC
            pltpu.sync_copy(src.at[pl.ds(off, _CC)], sidx)
            pltpu.sync_copy(dst.at[pl.ds(off, _CC)], didx)
            for v in range(_CC // 16):
                sadj[pl.ds(v * 16, 16)] = sidx[pl.ds(v * 16, 16)] + cbase
            pltpu.async_copy(ht.at[sadj], hrows, sem).wait()
            pltpu.async_copy(stab.at[didx], srows, sem).wait()
            pltpu.sync_copy(pfl.at[pl.ds(off * 16, _CC * 16)], pbuf)

            def edge(j, cc):
                w = pbuf[pl.ds(j * 16, 16)] * srows[j, pl.ds(0, 16)]
                acc = [zero16] * 4
                for h in range(8):
                    whb = jnp.full((16,), w[h], jnp.float32)
                    for v in range(4):
                        hv = hrows[j, pl.ds(h * 64 + v * 16, 16)]
                        acc[v] = acc[v] + whb * hv
                for v in range(4):
                    msg[j, pl.ds(v * 16, 16)] = acc[v]
                return cc

            lax.fori_loop(0, _CC, edge, 0)
            pltpu.sync_copy(msg, sacc.at[didx], add=True)
            return carry

        lax.fori_loop(0, _NCHC, chunk, 0)
        plsc.subcore_barrier()

        def fcp(i, cc):
            off = s * stripe + i * 128
            fill_zidx(off)
            pltpu.sync_copy(sacc.at[zidx], zbuf)
            pltpu.sync_copy(zbuf, out.at[c, pl.ds(off, 128)])
            return cc

        lax.fori_loop(0, stripe // 128, fcp, 0)

    return functools.partial(
        pl.kernel,
        out_type=[jax.ShapeDtypeStruct((2, _NP, 64), jnp.float32)],
        mesh=_MESH,
        scratch_types=[
            pltpu.VMEM((_CC,), jnp.int32),
            pltpu.VMEM((_CC,), jnp.int32),
            pltpu.VMEM((_CC,), jnp.int32),
            pltpu.VMEM((_CC, 512), jnp.float32),
            pltpu.VMEM((_CC, 128), jnp.float32),
            pltpu.VMEM((_CC * 16,), jnp.float32),
            pltpu.VMEM((_CC, 64), jnp.float32),
            pltpu.VMEM((128, 64), jnp.float32),
            pltpu.VMEM((128,), jnp.int32),
            pltpu.VMEM_SHARED((_NP, 64), jnp.float32),
            pltpu.SemaphoreType.DMA,
        ],
    )(_msg_body)


_msg0 = _make_msg(0)
_msg1 = _make_msg(1)


_attn = functools.partial(
    pl.kernel,
    out_type=[
        jax.ShapeDtypeStruct((_EP * 16,), jnp.float32),
        jax.ShapeDtypeStruct((2, _NP, 16), jnp.float32),
    ],
    mesh=_MESH,
    scratch_types=[
        pltpu.VMEM((_CB,), jnp.int32),
        pltpu.VMEM((_CB,), jnp.int32),
        pltpu.VMEM((_CB, 128), jnp.float32),
        pltpu.VMEM((_CB, 128), jnp.float32),
        pltpu.VMEM((_CB, 16), jnp.float32),
        pltpu.VMEM((_CB * 16,), jnp.float32),
        pltpu.VMEM((_CB, 16), jnp.float32),
        pltpu.VMEM((_CB,), jnp.int32),
        pltpu.VMEM_SHARED((_NP, 16), jnp.float32),
        pltpu.SemaphoreType.DMA,
    ],
)(_attn_body)


def _gat_conv(x, src, dst, srcp, dstp, z16, W, att_src, att_dst, bias):
    n = x.shape[0]
    heads, oc = att_src.shape
    h = _matmul(x, W).reshape(n, heads, oc)
    a_src = (h * att_src[None, :, :]).sum(-1)
    a_dst = (h * att_dst[None, :, :]).sum(-1)
    asd = jnp.zeros((_NP, 128), jnp.float32)
    asd = asd.at[:n, :8].set(a_src).at[:n, 8:16].set(a_dst)
    dsa = jnp.zeros((_NP, 128), jnp.float32)
    dsa = dsa.at[:n, :8].set(a_dst).at[:n, 8:16].set(a_src)
    p_flat, s_part = _attn(asd, dsa, srcp, dstp, z16)
    s = s_part[0, :, :8] + s_part[1, :, :8]
    sinv = 1.0 / (8.0 * (s + 1e-16))
    stab = jnp.zeros((_NP, 128), jnp.float32).at[:, :8].set(sinv)
    ht = jnp.zeros((4 * _NP, 512), jnp.float32)
    for q in range(4):
        ht = ht.at[q * _NP:q * _NP + n].set(
            h[:, :, q * 64:(q + 1) * 64].reshape(n, 512))
    (outa,) = _msg0(ht, stab, p_flat, srcp, dstp)
    (outb,) = _msg1(ht, stab, p_flat, srcp, dstp)
    return jnp.concatenate(
        [outa[0, :n], outb[0, :n], outa[1, :n], outb[1, :n]], axis=1) + bias


def kernel(x, edge_index, batch, W1, att_src1, att_dst1, b1,
           W2, att_src2, att_dst2, b2, Wl, bl):
    n = x.shape[0]
    loops = jnp.arange(n, dtype=edge_index.dtype)
    src = jnp.concatenate([edge_index[0], loops])
    dst = jnp.concatenate([edge_index[1], loops])
    pad = jnp.full((_EP - _E_TOT,), _N, dtype=edge_index.dtype)
    srcp = jnp.concatenate([src, pad])
    dstp = jnp.concatenate([dst, pad])
    z16 = jnp.zeros((_NP, 16), jnp.float32)
    h = jax.nn.relu(
        _gat_conv(x, src, dst, srcp, dstp, z16, W1, att_src1, att_dst1, b1))
    h = _gat_conv(h, src, dst, srcp, dstp, z16, W2, att_src2, att_dst2, b2)
    counts = jax.ops.segment_sum(jnp.ones((n,), h.dtype), batch,
                                 num_segments=64)
    pooled = jax.ops.segment_sum(h, batch, num_segments=64)
    pooled = pooled / jnp.clip(counts, 1.0)[:, None]
    return pooled @ Wl + bl


# speedup vs baseline: 9.7613x; 1.5906x over previous
import functools

import jax
import jax.numpy as jnp
from jax import lax
from jax.experimental import pallas as pl
from jax.experimental.pallas import tpu as pltpu
from jax.experimental.pallas import tpu_sc as plsc

_N = 10000
_NP = 10240
_E_TOT = 330000
_EP = 335872
_EPT = _EP // 32
_CB = 128
_NCH = _EPT // _CB

_EPP = _EP + 128
_EPT2 = _EP // 16
_CC = 64
_NCHC = _EPT2 // _CC

_MESH = plsc.VectorSubcoreMesh(
    core_axis_name="c", subcore_axis_name="s", num_cores=2, num_subcores=16)


def _mm_kernel(x_ref, w_ref, o_ref):
    o_ref[...] = jnp.dot(x_ref[...], w_ref[...],
                         preferred_element_type=jnp.float32)


def _matmul(x, w, blk_m=400):
    m, k = x.shape
    _, n = w.shape
    return pl.pallas_call(
        _mm_kernel,
        grid=(m // blk_m,),
        in_specs=[
            pl.BlockSpec((blk_m, k), lambda i: (i, 0)),
            pl.BlockSpec((k, n), lambda i: (0, 0)),
        ],
        out_specs=pl.BlockSpec((blk_m, n), lambda i: (i, 0)),
        out_shape=jax.ShapeDtypeStruct((m, n), jnp.float32),
    )(x, w)


def _attn_body(asd, dsa, src, dst, z16, p_out, s_out,
               sidx, didx, srows, drows, pbuf, pflat, zbuf, zidx, sacc, sem):
    c = lax.axis_index("c")
    s = lax.axis_index("s")
    wid = s * 2 + c
    stripe = _NP // 16

    def zrow(j, cc):
        zbuf[j] = jnp.zeros((16,), jnp.float32)
        return cc

    lax.fori_loop(0, _CB, zrow, 0)

    def fill_zidx(off):
        iot = lax.iota(jnp.int32, 16)
        for j in range(_CB // 16):
            zidx[pl.ds(j * 16, 16)] = iot + (off + j * 16)

    def zcp(i, cc):
        fill_zidx(s * stripe + i * _CB)
        pltpu.sync_copy(zbuf, sacc.at[zidx])
        return cc

    lax.fori_loop(0, stripe // _CB, zcp, 0)
    plsc.subcore_barrier()
    base = wid * _EPT

    def chunk(k, carry):
        off = base + k * _CB
        pltpu.sync_copy(src.at[pl.ds(off, _CB)], sidx)
        pltpu.sync_copy(dst.at[pl.ds(off, _CB)], didx)
        pltpu.async_copy(asd.at[sidx], srows, sem).wait()
        pltpu.async_copy(dsa.at[didx], drows, sem).wait()

        def edge(j, cc):
            e = srows[j, pl.ds(0, 16)] + drows[j, pl.ds(0, 16)]
            e = jnp.where(e > 0, e, 0.2 * e)
            p = jnp.exp(e)
            pbuf[j] = p
            pflat[pl.ds(j * 16, 16)] = p
            return cc

        lax.fori_loop(0, _CB, edge, 0)
        pltpu.sync_copy(pflat, p_out.at[pl.ds(off * 16, _CB * 16)])
        pltpu.sync_copy(pbuf, sacc.at[didx], add=True)
        return carry

    lax.fori_loop(0, _NCH, chunk, 0)
    plsc.subcore_barrier()

    def fcp(i, cc):
        off = s * stripe + i * _CB
        fill_zidx(off)
        pltpu.sync_copy(sacc.at[zidx], zbuf)
        pltpu.sync_copy(zbuf, s_out.at[c, pl.ds(off, _CB)])
        return cc

    lax.fori_loop(0, stripe // _CB, fcp, 0)


def _make_msg(half):

    def _msg_body(ht, stab, pfl, src, dst, out,
                  sidx0, sidx1, didx0, didx1, sadj0, sadj1,
                  hrows0, hrows1, srows, pbuf0, pbuf1,
                  msg, zidx, sacc, sem0, sem1):
        c = lax.axis_index("c")
        s = lax.axis_index("s")
        stripe = _NP // 16
        zero16 = jnp.zeros((16,), jnp.float32)
        sidx = (sidx0, sidx1)
        didx = (didx0, didx1)
        sadj = (sadj0, sadj1)
        hrows = (hrows0, hrows1)
        pbuf = (pbuf0, pbuf1)
        sem = (sem0, sem1)

        def zrow(j, cc):
            for v in range(4):
                msg[j, pl.ds(v * 16, 16)] = zero16
            return cc

        lax.fori_loop(0, 64, zrow, 0)

        def fill_zidx(off):
            iot = lax.iota(jnp.int32, 16)
            for j in range(4):
                zidx[pl.ds(j * 16, 16)] = iot + (off + j * 16)

        def zcp(i, cc):
            fill_zidx(s * stripe + i * 64)
            pltpu.sync_copy(msg, sacc.at[zidx])
            return cc

        lax.fori_loop(0, stripe // 64, zcp, 0)
        plsc.subcore_barrier()
        base = s * _EPT2
        cbase = (c * 2 + half) * _NP

        def fetch(b, k):
            off = base + k * _CC
            pltpu.sync_copy(src.at[pl.ds(off, _CC)], sidx[b])
            pltpu.sync_copy(dst.at[pl.ds(off, _CC)], didx[b])
            for v in range(_CC // 16):
                sadj[b][pl.ds(v * 16, 16)] = (
                    sidx[b][pl.ds(v * 16, 16)] + cbase)
            pltpu.async_copy(ht.at[sadj[b]], hrows[b], sem[b])
            pltpu.async_copy(pfl.at[pl.ds(off * 16, _CC * 16)],
                             pbuf[b], sem[b])

        def wait(b, k):
            off = base + k * _CC
            pltpu.make_async_copy(ht.at[sadj[b]], hrows[b], sem[b]).wait()
            pltpu.make_async_copy(pfl.at[pl.ds(off * 16, _CC * 16)],
                                  pbuf[b], sem[b]).wait()

        def compute(b):
            pltpu.async_copy(stab.at[didx[b]], srows, sem[b]).wait()

            def edge(j, cc):
                w = pbuf[b][pl.ds(j * 16, 16)] * srows[j, pl.ds(0, 16)]
                acc = [zero16] * 4
                for h in range(8):
                    whb = jnp.full((16,), w[h], jnp.float32)
                    for v in range(4):
                        hv = hrows[b][j, pl.ds(h * 64 + v * 16, 16)]
                        acc[v] = acc[v] + whb * hv
                for v in range(4):
                    msg[j, pl.ds(v * 16, 16)] = acc[v]
                return cc

            lax.fori_loop(0, _CC, edge, 0)
            pltpu.sync_copy(msg, sacc.at[didx[b]], add=True)

        for b in range(2):
            fetch(b, b)

        def outer(t, carry):
            for b in range(2):
                k = t * 2 + b
                wait(b, k)
                compute(b)
                fetch(b, k + 2)
            return carry

        lax.fori_loop(0, _NCHC // 2, outer, 0)
        for b in range(2):
            wait(b, _NCHC + b)
        plsc.subcore_barrier()

        def fcp(i, cc):
            off = s * stripe + i * 64
            fill_zidx(off)
            pltpu.sync_copy(sacc.at[zidx], msg)
            pltpu.sync_copy(msg, out.at[c, pl.ds(off, 64)])
            return cc

        lax.fori_loop(0, stripe // 64, fcp, 0)

    return functools.partial(
        pl.kernel,
        out_type=[jax.ShapeDtypeStruct((2, _NP, 64), jnp.float32)],
        mesh=_MESH,
        scratch_types=(
            [pltpu.VMEM((_CC,), jnp.int32)] * 6
            + [pltpu.VMEM((_CC, 512), jnp.float32)] * 2
            + [pltpu.VMEM((_CC, 128), jnp.float32)]
            + [pltpu.VMEM((_CC * 16,), jnp.float32)] * 2
            + [
                pltpu.VMEM((_CC, 64), jnp.float32),
                pltpu.VMEM((64,), jnp.int32),
                pltpu.VMEM_SHARED((_NP, 64), jnp.float32),
                pltpu.SemaphoreType.DMA,
                pltpu.SemaphoreType.DMA,
            ]
        ),
    )(_msg_body)


_msg0 = _make_msg(0)
_msg1 = _make_msg(1)


_attn = functools.partial(
    pl.kernel,
    out_type=[
        jax.ShapeDtypeStruct((_EPP * 16,), jnp.float32),
        jax.ShapeDtypeStruct((2, _NP, 16), jnp.float32),
    ],
    mesh=_MESH,
    scratch_types=[
        pltpu.VMEM((_CB,), jnp.int32),
        pltpu.VMEM((_CB,), jnp.int32),
        pltpu.VMEM((_CB, 128), jnp.float32),
        pltpu.VMEM((_CB, 128), jnp.float32),
        pltpu.VMEM((_CB, 16), jnp.float32),
        pltpu.VMEM((_CB * 16,), jnp.float32),
        pltpu.VMEM((_CB, 16), jnp.float32),
        pltpu.VMEM((_CB,), jnp.int32),
        pltpu.VMEM_SHARED((_NP, 16), jnp.float32),
        pltpu.SemaphoreType.DMA,
    ],
)(_attn_body)


def _gat_conv(x, src, dst, srcp, dstp, z16, W, att_src, att_dst, bias):
    n = x.shape[0]
    heads, oc = att_src.shape
    h = _matmul(x, W).reshape(n, heads, oc)
    a_src = (h * att_src[None, :, :]).sum(-1)
    a_dst = (h * att_dst[None, :, :]).sum(-1)
    asd = jnp.zeros((_NP, 128), jnp.float32)
    asd = asd.at[:n, :8].set(a_src).at[:n, 8:16].set(a_dst)
    dsa = jnp.zeros((_NP, 128), jnp.float32)
    dsa = dsa.at[:n, :8].set(a_dst).at[:n, 8:16].set(a_src)
    p_flat, s_part = _attn(asd, dsa, srcp, dstp, z16)
    s = s_part[0, :, :8] + s_part[1, :, :8]
    sinv = 1.0 / (8.0 * (s + 1e-16))
    stab = jnp.zeros((_NP, 128), jnp.float32).at[:, :8].set(sinv)
    ht = jnp.zeros((4 * _NP, 512), jnp.float32)
    for q in range(4):
        ht = ht.at[q * _NP:q * _NP + n].set(
            h[:, :, q * 64:(q + 1) * 64].reshape(n, 512))
    (outa,) = _msg0(ht, stab, p_flat, srcp, dstp)
    (outb,) = _msg1(ht, stab, p_flat, srcp, dstp)
    return jnp.concatenate(
        [outa[0, :n], outb[0, :n], outa[1, :n], outb[1, :n]], axis=1) + bias


def kernel(x, edge_index, batch, W1, att_src1, att_dst1, b1,
           W2, att_src2, att_dst2, b2, Wl, bl):
    n = x.shape[0]
    loops = jnp.arange(n, dtype=edge_index.dtype)
    src = jnp.concatenate([edge_index[0], loops])
    dst = jnp.concatenate([edge_index[1], loops])
    pad = jnp.full((_EPP - _E_TOT,), _N, dtype=edge_index.dtype)
    srcp = jnp.concatenate([src, pad])
    dstp = jnp.concatenate([dst, pad])
    z16 = jnp.zeros((_NP, 16), jnp.float32)
    h = jax.nn.relu(
        _gat_conv(x, src, dst, srcp, dstp, z16, W1, att_src1, att_dst1, b1))
    h = _gat_conv(h, src, dst, srcp, dstp, z16, W2, att_src2, att_dst2, b2)
    counts = jax.ops.segment_sum(jnp.ones((n,), h.dtype), batch,
                                 num_segments=64)
    pooled = jax.ops.segment_sum(h, batch, num_segments=64)
    pooled = pooled / jnp.clip(counts, 1.0)[:, None]
    return pooled @ Wl + bl


# speedup vs baseline: 11.9564x; 1.2249x over previous
import functools

import jax
import jax.numpy as jnp
from jax import lax
from jax.experimental import pallas as pl
from jax.experimental.pallas import tpu as pltpu
from jax.experimental.pallas import tpu_sc as plsc

_N = 10000
_NP = 10240
_E_TOT = 330000
_EP = 335872
_EPT = _EP // 32
_CB = 128
_NCH = _EPT // _CB

_EPP = _EP + 256
_EPT2 = _EP // 16
_CC = 64
_NCHC = _EPT2 // _CC

_MESH = plsc.VectorSubcoreMesh(
    core_axis_name="c", subcore_axis_name="s", num_cores=2, num_subcores=16)


def _mm_kernel(x_ref, w_ref, o_ref):
    o_ref[...] = jnp.dot(x_ref[...], w_ref[...],
                         preferred_element_type=jnp.float32)


def _matmul(x, w, blk_m=400):
    m, k = x.shape
    _, n = w.shape
    return pl.pallas_call(
        _mm_kernel,
        grid=(m // blk_m,),
        in_specs=[
            pl.BlockSpec((blk_m, k), lambda i: (i, 0)),
            pl.BlockSpec((k, n), lambda i: (0, 0)),
        ],
        out_specs=pl.BlockSpec((blk_m, n), lambda i: (i, 0)),
        out_shape=jax.ShapeDtypeStruct((m, n), jnp.float32),
    )(x, w)


def _attn_body(asd, dsa, src, dst, z16, p_out, s_out,
               sidx, didx, srows, drows, pbuf, pflat, zbuf, zidx, sacc, sem):
    c = lax.axis_index("c")
    s = lax.axis_index("s")
    wid = s * 2 + c
    stripe = _NP // 16

    def zrow(j, cc):
        zbuf[j] = jnp.zeros((16,), jnp.float32)
        return cc

    lax.fori_loop(0, _CB, zrow, 0)

    def fill_zidx(off):
        iot = lax.iota(jnp.int32, 16)
        for j in range(_CB // 16):
            zidx[pl.ds(j * 16, 16)] = iot + (off + j * 16)

    def zcp(i, cc):
        fill_zidx(s * stripe + i * _CB)
        pltpu.sync_copy(zbuf, sacc.at[zidx])
        return cc

    lax.fori_loop(0, stripe // _CB, zcp, 0)
    plsc.subcore_barrier()
    base = wid * _EPT

    def chunk(k, carry):
        off = base + k * _CB
        pltpu.sync_copy(src.at[pl.ds(off, _CB)], sidx)
        pltpu.sync_copy(dst.at[pl.ds(off, _CB)], didx)
        pltpu.async_copy(asd.at[sidx], srows, sem).wait()
        pltpu.async_copy(dsa.at[didx], drows, sem).wait()

        def edge(j, cc):
            e = srows[j, pl.ds(0, 16)] + drows[j, pl.ds(0, 16)]
            e = jnp.where(e > 0, e, 0.2 * e)
            p = jnp.exp(e)
            pbuf[j] = p
            pflat[pl.ds(j * 16, 16)] = p
            return cc

        lax.fori_loop(0, _CB, edge, 0)
        pltpu.sync_copy(pflat, p_out.at[pl.ds(off * 16, _CB * 16)])
        pltpu.sync_copy(pbuf, sacc.at[didx], add=True)
        return carry

    lax.fori_loop(0, _NCH, chunk, 0)
    plsc.subcore_barrier()

    def fcp(i, cc):
        off = s * stripe + i * _CB
        fill_zidx(off)
        pltpu.sync_copy(sacc.at[zidx], zbuf)
        pltpu.sync_copy(zbuf, s_out.at[c, pl.ds(off, _CB)])
        return cc

    lax.fori_loop(0, stripe // _CB, fcp, 0)


def _norm_body(stab, pfl, dst, w_out,
               didx0, didx1, srows0, srows1, pbuf0, pbuf1, wbuf,
               sem0, sem1):
    c = lax.axis_index("c")
    s = lax.axis_index("s")
    wid = s * 2 + c
    base = wid * _EPT
    didx = (didx0, didx1)
    srows = (srows0, srows1)
    pbuf = (pbuf0, pbuf1)
    sem = (sem0, sem1)

    def fetch(b, k):
        off = base + k * _CB
        pltpu.sync_copy(dst.at[pl.ds(off, _CB)], didx[b])
        pltpu.async_copy(stab.at[didx[b]], srows[b], sem[b])
        pltpu.async_copy(pfl.at[pl.ds(off * 16, _CB * 16)], pbuf[b], sem[b])

    def wait(b, k):
        off = base + k * _CB
        pltpu.make_async_copy(stab.at[didx[b]], srows[b], sem[b]).wait()
        pltpu.make_async_copy(pfl.at[pl.ds(off * 16, _CB * 16)],
                              pbuf[b], sem[b]).wait()

    for b in range(2):
        fetch(b, b)

    def outer(t, carry):
        for b in range(2):
            k = t * 2 + b
            wait(b, k)

            def edge(j, cc):
                w = pbuf[b][pl.ds(j * 16, 16)] * srows[b][j, pl.ds(0, 16)]
                wbuf[pl.ds(j * 16, 16)] = w
                return cc

            lax.fori_loop(0, _CB, edge, 0)
            off = base + k * _CB
            pltpu.sync_copy(wbuf, w_out.at[pl.ds(off * 16, _CB * 16)])
            fetch(b, k + 2)
        return carry

    lax.fori_loop(0, _NCH // 2, outer, 0)
    for b in range(2):
        wait(b, _NCH + b)


_norm = functools.partial(
    pl.kernel,
    out_type=[jax.ShapeDtypeStruct((_EPP * 16,), jnp.float32)],
    mesh=_MESH,
    scratch_types=[
        pltpu.VMEM((_CB,), jnp.int32),
        pltpu.VMEM((_CB,), jnp.int32),
        pltpu.VMEM((_CB, 128), jnp.float32),
        pltpu.VMEM((_CB, 128), jnp.float32),
        pltpu.VMEM((_CB * 16,), jnp.float32),
        pltpu.VMEM((_CB * 16,), jnp.float32),
        pltpu.VMEM((_CB * 16,), jnp.float32),
        pltpu.SemaphoreType.DMA,
        pltpu.SemaphoreType.DMA,
    ],
)(_norm_body)


def _make_msg(half):

    def _msg_body(ht, pfl, src, dst, out,
                  sidx0, sidx1, didx0, didx1, sadj0, sadj1,
                  hrows0, hrows1, pbuf0, pbuf1,
                  msg, zidx, sacc, sem0, sem1):
        c = lax.axis_index("c")
        s = lax.axis_index("s")
        stripe = _NP // 16
        zero16 = jnp.zeros((16,), jnp.float32)
        sidx = (sidx0, sidx1)
        didx = (didx0, didx1)
        sadj = (sadj0, sadj1)
        hrows = (hrows0, hrows1)
        pbuf = (pbuf0, pbuf1)
        sem = (sem0, sem1)

        def zrow(j, cc):
            for v in range(4):
                msg[j, pl.ds(v * 16, 16)] = zero16
            return cc

        lax.fori_loop(0, 64, zrow, 0)

        def fill_zidx(off):
            iot = lax.iota(jnp.int32, 16)
            for j in range(4):
                zidx[pl.ds(j * 16, 16)] = iot + (off + j * 16)

        def zcp(i, cc):
            fill_zidx(s * stripe + i * 64)
            pltpu.sync_copy(msg, sacc.at[zidx])
            return cc

        lax.fori_loop(0, stripe // 64, zcp, 0)
        plsc.subcore_barrier()
        base = s * _EPT2
        cbase = (c * 2 + half) * _NP

        def fetch(b, k):
            off = base + k * _CC
            pltpu.sync_copy(src.at[pl.ds(off, _CC)], sidx[b])
            pltpu.sync_copy(dst.at[pl.ds(off, _CC)], didx[b])
            for v in range(_CC // 16):
                sadj[b][pl.ds(v * 16, 16)] = (
                    sidx[b][pl.ds(v * 16, 16)] + cbase)
            pltpu.async_copy(ht.at[sadj[b]], hrows[b], sem[b])
            pltpu.async_copy(pfl.at[pl.ds(off * 16, _CC * 16)],
                             pbuf[b], sem[b])

        def wait(b, k):
            off = base + k * _CC
            pltpu.make_async_copy(ht.at[sadj[b]], hrows[b], sem[b]).wait()
            pltpu.make_async_copy(pfl.at[pl.ds(off * 16, _CC * 16)],
                                  pbuf[b], sem[b]).wait()

        def compute(b):
            def edge(j, cc):
                w = pbuf[b][pl.ds(j * 16, 16)]
                acc = [zero16] * 4
                for h in range(8):
                    whb = jnp.full((16,), w[h], jnp.float32)
                    for v in range(4):
                        hv = hrows[b][j, pl.ds(h * 64 + v * 16, 16)]
                        acc[v] = acc[v] + whb * hv
                for v in range(4):
                    msg[j, pl.ds(v * 16, 16)] = acc[v]
                return cc

            lax.fori_loop(0, _CC, edge, 0)
            pltpu.sync_copy(msg, sacc.at[didx[b]], add=True)

        for b in range(2):
            fetch(b, b)

        def outer(t, carry):
            for b in range(2):
                k = t * 2 + b
                wait(b, k)
                compute(b)
                fetch(b, k + 2)
            return carry

        lax.fori_loop(0, _NCHC // 2, outer, 0)
        for b in range(2):
            wait(b, _NCHC + b)
        plsc.subcore_barrier()

        def fcp(i, cc):
            off = s * stripe + i * 64
            fill_zidx(off)
            pltpu.sync_copy(sacc.at[zidx], msg)
            pltpu.sync_copy(msg, out.at[c, pl.ds(off, 64)])
            return cc

        lax.fori_loop(0, stripe // 64, fcp, 0)

    return functools.partial(
        pl.kernel,
        out_type=[jax.ShapeDtypeStruct((2, _NP, 64), jnp.float32)],
        mesh=_MESH,
        scratch_types=(
            [pltpu.VMEM((_CC,), jnp.int32)] * 6
            + [pltpu.VMEM((_CC, 512), jnp.float32)] * 2
            + [pltpu.VMEM((_CC * 16,), jnp.float32)] * 2
            + [
                pltpu.VMEM((_CC, 64), jnp.float32),
                pltpu.VMEM((64,), jnp.int32),
                pltpu.VMEM_SHARED((_NP, 64), jnp.float32),
                pltpu.SemaphoreType.DMA,
                pltpu.SemaphoreType.DMA,
            ]
        ),
    )(_msg_body)


_msg0 = _make_msg(0)
_msg1 = _make_msg(1)


_attn = functools.partial(
    pl.kernel,
    out_type=[
        jax.ShapeDtypeStruct((_EPP * 16,), jnp.float32),
        jax.ShapeDtypeStruct((2, _NP, 16), jnp.float32),
    ],
    mesh=_MESH,
    scratch_types=[
        pltpu.VMEM((_CB,), jnp.int32),
        pltpu.VMEM((_CB,), jnp.int32),
        pltpu.VMEM((_CB, 128), jnp.float32),
        pltpu.VMEM((_CB, 128), jnp.float32),
        pltpu.VMEM((_CB, 16), jnp.float32),
        pltpu.VMEM((_CB * 16,), jnp.float32),
        pltpu.VMEM((_CB, 16), jnp.float32),
        pltpu.VMEM((_CB,), jnp.int32),
        pltpu.VMEM_SHARED((_NP, 16), jnp.float32),
        pltpu.SemaphoreType.DMA,
    ],
)(_attn_body)


def _gat_conv(x, src, dst, srcp, dstp, z16, W, att_src, att_dst, bias):
    n = x.shape[0]
    heads, oc = att_src.shape
    h = _matmul(x, W).reshape(n, heads, oc)
    a_src = (h * att_src[None, :, :]).sum(-1)
    a_dst = (h * att_dst[None, :, :]).sum(-1)
    asd = jnp.zeros((_NP, 128), jnp.float32)
    asd = asd.at[:n, :8].set(a_src).at[:n, 8:16].set(a_dst)
    dsa = jnp.zeros((_NP, 128), jnp.float32)
    dsa = dsa.at[:n, :8].set(a_dst).at[:n, 8:16].set(a_src)
    p_flat, s_part = _attn(asd, dsa, srcp, dstp, z16)
    s = s_part[0, :, :8] + s_part[1, :, :8]
    sinv = 1.0 / (8.0 * (s + 1e-16))
    stab = jnp.zeros((_NP, 128), jnp.float32).at[:, :8].set(sinv)
    ht = jnp.zeros((4 * _NP, 512), jnp.float32)
    for q in range(4):
        ht = ht.at[q * _NP:q * _NP + n].set(
            h[:, :, q * 64:(q + 1) * 64].reshape(n, 512))
    (w_flat,) = _norm(stab, p_flat, dstp)
    (outa,) = _msg0(ht, w_flat, srcp, dstp)
    (outb,) = _msg1(ht, w_flat, srcp, dstp)
    return jnp.concatenate(
        [outa[0, :n], outb[0, :n], outa[1, :n], outb[1, :n]], axis=1) + bias


def kernel(x, edge_index, batch, W1, att_src1, att_dst1, b1,
           W2, att_src2, att_dst2, b2, Wl, bl):
    n = x.shape[0]
    loops = jnp.arange(n, dtype=edge_index.dtype)
    src = jnp.concatenate([edge_index[0], loops])
    dst = jnp.concatenate([edge_index[1], loops])
    pad = jnp.full((_EPP - _E_TOT,), _N, dtype=edge_index.dtype)
    srcp = jnp.concatenate([src, pad])
    dstp = jnp.concatenate([dst, pad])
    z16 = jnp.zeros((_NP, 16), jnp.float32)
    h = jax.nn.relu(
        _gat_conv(x, src, dst, srcp, dstp, z16, W1, att_src1, att_dst1, b1))
    h = _gat_conv(h, src, dst, srcp, dstp, z16, W2, att_src2, att_dst2, b2)
    counts = jax.ops.segment_sum(jnp.ones((n,), h.dtype), batch,
                                 num_segments=64)
    pooled = jax.ops.segment_sum(h, batch, num_segments=64)
    pooled = pooled / jnp.clip(counts, 1.0)[:, None]
    return pooled @ Wl + bl


# speedup vs baseline: 13.3073x; 1.1130x over previous
import functools

import jax
import jax.numpy as jnp
from jax import lax
from jax.experimental import pallas as pl
from jax.experimental.pallas import tpu as pltpu
from jax.experimental.pallas import tpu_sc as plsc

_N = 10000
_NP = 10240
_E_TOT = 330000
_EP = 335872
_EPT = _EP // 32
_CB = 128
_NCH = _EPT // _CB

_EPP = _EP + 256
_EPT2 = _EP // 16
_CC = 64
_NCHC = _EPT2 // _CC

_MESH = plsc.VectorSubcoreMesh(
    core_axis_name="c", subcore_axis_name="s", num_cores=2, num_subcores=16)


def _mm_kernel(x_ref, w_ref, o_ref):
    o_ref[...] = jnp.dot(x_ref[...], w_ref[...],
                         preferred_element_type=jnp.float32)


def _matmul(x, w, blk_m=400):
    m, k = x.shape
    _, n = w.shape
    return pl.pallas_call(
        _mm_kernel,
        grid=(m // blk_m,),
        in_specs=[
            pl.BlockSpec((blk_m, k), lambda i: (i, 0)),
            pl.BlockSpec((k, n), lambda i: (0, 0)),
        ],
        out_specs=pl.BlockSpec((blk_m, n), lambda i: (i, 0)),
        out_shape=jax.ShapeDtypeStruct((m, n), jnp.float32),
    )(x, w)


def _attn_body(asd, dsa, src, dst, z16, p_out, s_out,
               sidx0, sidx1, didx0, didx1, srows0, srows1, drows0, drows1,
               pbuf, pflat, zbuf, zidx, sacc, sem0, sem1):
    c = lax.axis_index("c")
    s = lax.axis_index("s")
    wid = s * 2 + c
    stripe = _NP // 16
    sidx = (sidx0, sidx1)
    didx = (didx0, didx1)
    srows = (srows0, srows1)
    drows = (drows0, drows1)
    sem = (sem0, sem1)

    def zrow(j, cc):
        zbuf[j] = jnp.zeros((16,), jnp.float32)
        return cc

    lax.fori_loop(0, _CB, zrow, 0)

    def fill_zidx(off):
        iot = lax.iota(jnp.int32, 16)
        for j in range(_CB // 16):
            zidx[pl.ds(j * 16, 16)] = iot + (off + j * 16)

    def zcp(i, cc):
        fill_zidx(s * stripe + i * _CB)
        pltpu.sync_copy(zbuf, sacc.at[zidx])
        return cc

    lax.fori_loop(0, stripe // _CB, zcp, 0)
    plsc.subcore_barrier()
    base = wid * _EPT

    def fetch(b, k):
        off = base + k * _CB
        pltpu.sync_copy(src.at[pl.ds(off, _CB)], sidx[b])
        pltpu.sync_copy(dst.at[pl.ds(off, _CB)], didx[b])
        pltpu.async_copy(asd.at[sidx[b]], srows[b], sem[b])
        pltpu.async_copy(dsa.at[didx[b]], drows[b], sem[b])

    def wait(b):
        pltpu.make_async_copy(asd.at[sidx[b]], srows[b], sem[b]).wait()
        pltpu.make_async_copy(dsa.at[didx[b]], drows[b], sem[b]).wait()

    for b in range(2):
        fetch(b, b)

    def outer(t, carry):
        for b in range(2):
            k = t * 2 + b
            wait(b)

            def edge(j, cc):
                e = srows[b][j, pl.ds(0, 16)] + drows[b][j, pl.ds(0, 16)]
                e = jnp.where(e > 0, e, 0.2 * e)
                p = jnp.exp(e)
                pbuf[j] = p
                pflat[pl.ds(j * 16, 16)] = p
                return cc

            lax.fori_loop(0, _CB, edge, 0)
            off = base + k * _CB
            pltpu.sync_copy(pflat, p_out.at[pl.ds(off * 16, _CB * 16)])
            pltpu.sync_copy(pbuf, sacc.at[didx[b]], add=True)
            fetch(b, k + 2)
        return carry

    lax.fori_loop(0, _NCH // 2, outer, 0)
    for b in range(2):
        wait(b)
    plsc.subcore_barrier()

    def fcp(i, cc):
        off = s * stripe + i * _CB
        fill_zidx(off)
        pltpu.sync_copy(sacc.at[zidx], zbuf)
        pltpu.sync_copy(zbuf, s_out.at[c, pl.ds(off, _CB)])
        return cc

    lax.fori_loop(0, stripe // _CB, fcp, 0)


def _norm_body(stab, pfl, dst, w_out,
               didx0, didx1, srows0, srows1, pbuf0, pbuf1, wbuf,
               sem0, sem1):
    c = lax.axis_index("c")
    s = lax.axis_index("s")
    wid = s * 2 + c
    base = wid * _EPT
    didx = (didx0, didx1)
    srows = (srows0, srows1)
    pbuf = (pbuf0, pbuf1)
    sem = (sem0, sem1)

    def fetch(b, k):
        off = base + k * _CB
        pltpu.sync_copy(dst.at[pl.ds(off, _CB)], didx[b])
        pltpu.async_copy(stab.at[didx[b]], srows[b], sem[b])
        pltpu.async_copy(pfl.at[pl.ds(off * 16, _CB * 16)], pbuf[b], sem[b])

    def wait(b, k):
        off = base + k * _CB
        pltpu.make_async_copy(stab.at[didx[b]], srows[b], sem[b]).wait()
        pltpu.make_async_copy(pfl.at[pl.ds(off * 16, _CB * 16)],
                              pbuf[b], sem[b]).wait()

    for b in range(2):
        fetch(b, b)

    def outer(t, carry):
        for b in range(2):
            k = t * 2 + b
            wait(b, k)

            def edge(j, cc):
                w = pbuf[b][pl.ds(j * 16, 16)] * srows[b][j, pl.ds(0, 16)]
                wbuf[pl.ds(j * 16, 16)] = w
                return cc

            lax.fori_loop(0, _CB, edge, 0)
            off = base + k * _CB
            pltpu.sync_copy(wbuf, w_out.at[pl.ds(off * 16, _CB * 16)])
            fetch(b, k + 2)
        return carry

    lax.fori_loop(0, _NCH // 2, outer, 0)
    for b in range(2):
        wait(b, _NCH + b)


_norm = functools.partial(
    pl.kernel,
    out_type=[jax.ShapeDtypeStruct((_EPP * 16,), jnp.float32)],
    mesh=_MESH,
    scratch_types=[
        pltpu.VMEM((_CB,), jnp.int32),
        pltpu.VMEM((_CB,), jnp.int32),
        pltpu.VMEM((_CB, 128), jnp.float32),
        pltpu.VMEM((_CB, 128), jnp.float32),
        pltpu.VMEM((_CB * 16,), jnp.float32),
        pltpu.VMEM((_CB * 16,), jnp.float32),
        pltpu.VMEM((_CB * 16,), jnp.float32),
        pltpu.SemaphoreType.DMA,
        pltpu.SemaphoreType.DMA,
    ],
)(_norm_body)


def _make_msg(half):

    def _msg_body(ht, pfl, src, dst, out,
                  sidx0, sidx1, didx0, didx1, sadj0, sadj1,
                  hrows0, hrows1, pbuf0, pbuf1,
                  msg, zidx, sacc, sem0, sem1):
        c = lax.axis_index("c")
        s = lax.axis_index("s")
        stripe = _NP // 16
        zero16 = jnp.zeros((16,), jnp.float32)
        sidx = (sidx0, sidx1)
        didx = (didx0, didx1)
        sadj = (sadj0, sadj1)
        hrows = (hrows0, hrows1)
        pbuf = (pbuf0, pbuf1)
        sem = (sem0, sem1)

        def zrow(j, cc):
            for v in range(4):
                msg[j, pl.ds(v * 16, 16)] = zero16
            return cc

        lax.fori_loop(0, 64, zrow, 0)

        def fill_zidx(off):
            iot = lax.iota(jnp.int32, 16)
            for j in range(4):
                zidx[pl.ds(j * 16, 16)] = iot + (off + j * 16)

        def zcp(i, cc):
            fill_zidx(s * stripe + i * 64)
            pltpu.sync_copy(msg, sacc.at[zidx])
            return cc

        lax.fori_loop(0, stripe // 64, zcp, 0)
        plsc.subcore_barrier()
        base = s * _EPT2
        cbase = (c * 2 + half) * _NP

        def fetch(b, k):
            off = base + k * _CC
            pltpu.sync_copy(src.at[pl.ds(off, _CC)], sidx[b])
            pltpu.sync_copy(dst.at[pl.ds(off, _CC)], didx[b])
            for v in range(_CC // 16):
                sadj[b][pl.ds(v * 16, 16)] = (
                    sidx[b][pl.ds(v * 16, 16)] + cbase)
            pltpu.async_copy(ht.at[sadj[b]], hrows[b], sem[b])
            pltpu.async_copy(pfl.at[pl.ds(off * 16, _CC * 16)],
                             pbuf[b], sem[b])

        def wait(b, k):
            off = base + k * _CC
            pltpu.make_async_copy(ht.at[sadj[b]], hrows[b], sem[b]).wait()
            pltpu.make_async_copy(pfl.at[pl.ds(off * 16, _CC * 16)],
                                  pbuf[b], sem[b]).wait()

        def compute(b):
            def edge(j, cc):
                w = pbuf[b][pl.ds(j * 16, 16)]
                acc = [zero16] * 4
                for h in range(8):
                    whb = jnp.full((16,), w[h], jnp.float32)
                    for v in range(4):
                        hv = hrows[b][j, pl.ds(h * 64 + v * 16, 16)]
                        acc[v] = acc[v] + whb * hv
                for v in range(4):
                    msg[j, pl.ds(v * 16, 16)] = acc[v]
                return cc

            lax.fori_loop(0, _CC, edge, 0)
            pltpu.sync_copy(msg, sacc.at[didx[b]], add=True)

        for b in range(2):
            fetch(b, b)

        def outer(t, carry):
            for b in range(2):
                k = t * 2 + b
                wait(b, k)
                compute(b)
                fetch(b, k + 2)
            return carry

        lax.fori_loop(0, _NCHC // 2, outer, 0)
        for b in range(2):
            wait(b, _NCHC + b)
        plsc.subcore_barrier()

        def fcp(i, cc):
            off = s * stripe + i * 64
            fill_zidx(off)
            pltpu.sync_copy(sacc.at[zidx], msg)
            pltpu.sync_copy(msg, out.at[c, pl.ds(off, 64)])
            return cc

        lax.fori_loop(0, stripe // 64, fcp, 0)

    return functools.partial(
        pl.kernel,
        out_type=[jax.ShapeDtypeStruct((2, _NP, 64), jnp.float32)],
        mesh=_MESH,
        scratch_types=(
            [pltpu.VMEM((_CC,), jnp.int32)] * 6
            + [pltpu.VMEM((_CC, 512), jnp.float32)] * 2
            + [pltpu.VMEM((_CC * 16,), jnp.float32)] * 2
            + [
                pltpu.VMEM((_CC, 64), jnp.float32),
                pltpu.VMEM((64,), jnp.int32),
                pltpu.VMEM_SHARED((_NP, 64), jnp.float32),
                pltpu.SemaphoreType.DMA,
                pltpu.SemaphoreType.DMA,
            ]
        ),
    )(_msg_body)


_msg0 = _make_msg(0)
_msg1 = _make_msg(1)


_attn = functools.partial(
    pl.kernel,
    out_type=[
        jax.ShapeDtypeStruct((_EPP * 16,), jnp.float32),
        jax.ShapeDtypeStruct((2, _NP, 16), jnp.float32),
    ],
    mesh=_MESH,
    scratch_types=(
        [pltpu.VMEM((_CB,), jnp.int32)] * 4
        + [pltpu.VMEM((_CB, 128), jnp.float32)] * 4
        + [
            pltpu.VMEM((_CB, 16), jnp.float32),
            pltpu.VMEM((_CB * 16,), jnp.float32),
            pltpu.VMEM((_CB, 16), jnp.float32),
            pltpu.VMEM((_CB,), jnp.int32),
            pltpu.VMEM_SHARED((_NP, 16), jnp.float32),
            pltpu.SemaphoreType.DMA,
            pltpu.SemaphoreType.DMA,
        ]
    ),
)(_attn_body)


def _gat_conv(x, src, dst, srcp, dstp, z16, W, att_src, att_dst, bias):
    n = x.shape[0]
    heads, oc = att_src.shape
    h = _matmul(x, W).reshape(n, heads, oc)
    a_src = (h * att_src[None, :, :]).sum(-1)
    a_dst = (h * att_dst[None, :, :]).sum(-1)
    asd = jnp.zeros((_NP, 128), jnp.float32)
    asd = asd.at[:n, :8].set(a_src).at[:n, 8:16].set(a_dst)
    dsa = jnp.zeros((_NP, 128), jnp.float32)
    dsa = dsa.at[:n, :8].set(a_dst).at[:n, 8:16].set(a_src)
    p_flat, s_part = _attn(asd, dsa, srcp, dstp, z16)
    s = s_part[0, :, :8] + s_part[1, :, :8]
    sinv = 1.0 / (8.0 * (s + 1e-16))
    stab = jnp.zeros((_NP, 128), jnp.float32).at[:, :8].set(sinv)
    ht = jnp.zeros((4 * _NP, 512), jnp.float32)
    for q in range(4):
        ht = ht.at[q * _NP:q * _NP + n].set(
            h[:, :, q * 64:(q + 1) * 64].reshape(n, 512))
    (w_flat,) = _norm(stab, p_flat, dstp)
    (outa,) = _msg0(ht, w_flat, srcp, dstp)
    (outb,) = _msg1(ht, w_flat, srcp, dstp)
    return jnp.concatenate(
        [outa[0, :n], outb[0, :n], outa[1, :n], outb[1, :n]], axis=1) + bias


def kernel(x, edge_index, batch, W1, att_src1, att_dst1, b1,
           W2, att_src2, att_dst2, b2, Wl, bl):
    n = x.shape[0]
    loops = jnp.arange(n, dtype=edge_index.dtype)
    src = jnp.concatenate([edge_index[0], loops])
    dst = jnp.concatenate([edge_index[1], loops])
    pad = jnp.full((_EPP - _E_TOT,), _N, dtype=edge_index.dtype)
    srcp = jnp.concatenate([src, pad])
    dstp = jnp.concatenate([dst, pad])
    z16 = jnp.zeros((_NP, 16), jnp.float32)
    h = jax.nn.relu(
        _gat_conv(x, src, dst, srcp, dstp, z16, W1, att_src1, att_dst1, b1))
    h = _gat_conv(h, src, dst, srcp, dstp, z16, W2, att_src2, att_dst2, b2)
    counts = jax.ops.segment_sum(jnp.ones((n,), h.dtype), batch,
                                 num_segments=64)
    pooled = jax.ops.segment_sum(h, batch, num_segments=64)
    pooled = pooled / jnp.clip(counts, 1.0)[:, None]
    return pooled @ Wl + bl


# speedup vs baseline: 13.9624x; 1.0492x over previous
import functools

import jax
import jax.numpy as jnp
from jax import lax
from jax.experimental import pallas as pl
from jax.experimental.pallas import tpu as pltpu
from jax.experimental.pallas import tpu_sc as plsc

_N = 10000
_NP = 10240
_E_TOT = 330000
_EP = 335872
_EPT = _EP // 32
_CB = 128
_NCH = _EPT // _CB

_EPP = _EP + 256
_EPT2 = _EP // 16
_CC = 64
_NCHC = _EPT2 // _CC

_MESH = plsc.VectorSubcoreMesh(
    core_axis_name="c", subcore_axis_name="s", num_cores=2, num_subcores=16)


def _mm_kernel(x_ref, w_ref, o_ref):
    o_ref[...] = jnp.dot(x_ref[...], w_ref[...],
                         preferred_element_type=jnp.float32)


def _matmul(x, w, blk_m=400):
    m, k = x.shape
    _, n = w.shape
    return pl.pallas_call(
        _mm_kernel,
        grid=(m // blk_m,),
        in_specs=[
            pl.BlockSpec((blk_m, k), lambda i: (i, 0)),
            pl.BlockSpec((k, n), lambda i: (0, 0)),
        ],
        out_specs=pl.BlockSpec((blk_m, n), lambda i: (i, 0)),
        out_shape=jax.ShapeDtypeStruct((m, n), jnp.float32),
    )(x, w)


def _attn_body(asd, dsa, src, dst, z16, p_out, s_out,
               sidx0, sidx1, didx0, didx1, srows0, srows1, drows0, drows1,
               pbuf, pflat, zbuf, zidx, sacc, sem0, sem1):
    c = lax.axis_index("c")
    s = lax.axis_index("s")
    wid = s * 2 + c
    stripe = _NP // 16
    sidx = (sidx0, sidx1)
    didx = (didx0, didx1)
    srows = (srows0, srows1)
    drows = (drows0, drows1)
    sem = (sem0, sem1)

    def zrow(j, cc):
        zbuf[j] = jnp.zeros((16,), jnp.float32)
        return cc

    lax.fori_loop(0, _CB, zrow, 0)

    def fill_zidx(off):
        iot = lax.iota(jnp.int32, 16)
        for j in range(_CB // 16):
            zidx[pl.ds(j * 16, 16)] = iot + (off + j * 16)

    def zcp(i, cc):
        fill_zidx(s * stripe + i * _CB)
        pltpu.sync_copy(zbuf, sacc.at[zidx])
        return cc

    lax.fori_loop(0, stripe // _CB, zcp, 0)
    plsc.subcore_barrier()
    base = wid * _EPT

    def fetch(b, k):
        off = base + k * _CB
        pltpu.sync_copy(src.at[pl.ds(off, _CB)], sidx[b])
        pltpu.sync_copy(dst.at[pl.ds(off, _CB)], didx[b])
        pltpu.async_copy(asd.at[sidx[b]], srows[b], sem[b])
        pltpu.async_copy(dsa.at[didx[b]], drows[b], sem[b])

    def wait(b):
        pltpu.make_async_copy(asd.at[sidx[b]], srows[b], sem[b]).wait()
        pltpu.make_async_copy(dsa.at[didx[b]], drows[b], sem[b]).wait()

    for b in range(2):
        fetch(b, b)

    def outer(t, carry):
        for b in range(2):
            k = t * 2 + b
            wait(b)

            def edge(j, cc):
                e = srows[b][j, pl.ds(0, 16)] + drows[b][j, pl.ds(0, 16)]
                e = jnp.where(e > 0, e, 0.2 * e)
                p = jnp.exp(e)
                pbuf[j] = p
                pflat[pl.ds(j * 16, 16)] = p
                return cc

            lax.fori_loop(0, _CB, edge, 0)
            off = base + k * _CB
            pltpu.sync_copy(pflat, p_out.at[pl.ds(off * 16, _CB * 16)])
            pltpu.sync_copy(pbuf, sacc.at[didx[b]], add=True)
            fetch(b, k + 2)
        return carry

    lax.fori_loop(0, _NCH // 2, outer, 0)
    for b in range(2):
        wait(b)
    plsc.subcore_barrier()

    def fcp(i, cc):
        off = s * stripe + i * _CB
        fill_zidx(off)
        pltpu.sync_copy(sacc.at[zidx], zbuf)
        pltpu.sync_copy(zbuf, s_out.at[c, pl.ds(off, _CB)])
        return cc

    lax.fori_loop(0, stripe // _CB, fcp, 0)


def _norm_body(stab, pfl, dst, w_out,
               didx0, didx1, srows0, srows1, pbuf0, pbuf1, wbuf,
               sem0, sem1):
    c = lax.axis_index("c")
    s = lax.axis_index("s")
    wid = s * 2 + c
    base = wid * _EPT
    didx = (didx0, didx1)
    srows = (srows0, srows1)
    pbuf = (pbuf0, pbuf1)
    sem = (sem0, sem1)

    def fetch(b, k):
        off = base + k * _CB
        pltpu.sync_copy(dst.at[pl.ds(off, _CB)], didx[b])
        pltpu.async_copy(stab.at[didx[b]], srows[b], sem[b])
        pltpu.async_copy(pfl.at[pl.ds(off * 16, _CB * 16)], pbuf[b], sem[b])

    def wait(b, k):
        off = base + k * _CB
        pltpu.make_async_copy(stab.at[didx[b]], srows[b], sem[b]).wait()
        pltpu.make_async_copy(pfl.at[pl.ds(off * 16, _CB * 16)],
                              pbuf[b], sem[b]).wait()

    for b in range(2):
        fetch(b, b)

    def outer(t, carry):
        for b in range(2):
            k = t * 2 + b
            wait(b, k)

            def edge(j, cc):
                w = pbuf[b][pl.ds(j * 16, 16)] * srows[b][j, pl.ds(0, 16)]
                wbuf[pl.ds(j * 16, 16)] = w
                return cc

            lax.fori_loop(0, _CB, edge, 0)
            off = base + k * _CB
            pltpu.sync_copy(wbuf, w_out.at[pl.ds(off * 16, _CB * 16)])
            fetch(b, k + 2)
        return carry

    lax.fori_loop(0, _NCH // 2, outer, 0)
    for b in range(2):
        wait(b, _NCH + b)


_norm = functools.partial(
    pl.kernel,
    out_type=[jax.ShapeDtypeStruct((_EPP * 16,), jnp.float32)],
    mesh=_MESH,
    scratch_types=[
        pltpu.VMEM((_CB,), jnp.int32),
        pltpu.VMEM((_CB,), jnp.int32),
        pltpu.VMEM((_CB, 128), jnp.float32),
        pltpu.VMEM((_CB, 128), jnp.float32),
        pltpu.VMEM((_CB * 16,), jnp.float32),
        pltpu.VMEM((_CB * 16,), jnp.float32),
        pltpu.VMEM((_CB * 16,), jnp.float32),
        pltpu.SemaphoreType.DMA,
        pltpu.SemaphoreType.DMA,
    ],
)(_norm_body)


def _make_msg(half):

    def _msg_body(ht, pfl, src, dst, out,
                  sidx0, sidx1, didx0, didx1, sadj0, sadj1,
                  hrows0, hrows1, pbuf0, pbuf1,
                  msg, zidx, sacc, sem0, sem1):
        c = lax.axis_index("c")
        s = lax.axis_index("s")
        stripe = _NP // 16
        zero16 = jnp.zeros((16,), jnp.float32)
        sidx = (sidx0, sidx1)
        didx = (didx0, didx1)
        sadj = (sadj0, sadj1)
        hrows = (hrows0, hrows1)
        pbuf = (pbuf0, pbuf1)
        sem = (sem0, sem1)

        def zrow(j, cc):
            for v in range(4):
                msg[j, pl.ds(v * 16, 16)] = zero16
            return cc

        lax.fori_loop(0, 64, zrow, 0)

        def fill_zidx(off):
            iot = lax.iota(jnp.int32, 16)
            for j in range(4):
                zidx[pl.ds(j * 16, 16)] = iot + (off + j * 16)

        def zcp(i, cc):
            fill_zidx(s * stripe + i * 64)
            pltpu.sync_copy(msg, sacc.at[zidx])
            return cc

        lax.fori_loop(0, stripe // 64, zcp, 0)
        plsc.subcore_barrier()
        base = s * _EPT2
        cbase = (c * 2 + half) * _NP

        def fetch(b, k):
            off = base + k * _CC
            pltpu.sync_copy(src.at[pl.ds(off, _CC)], sidx[b])
            pltpu.sync_copy(dst.at[pl.ds(off, _CC)], didx[b])
            for v in range(_CC // 16):
                sadj[b][pl.ds(v * 16, 16)] = (
                    sidx[b][pl.ds(v * 16, 16)] + cbase)
            pltpu.async_copy(ht.at[sadj[b]], hrows[b], sem[b])
            pltpu.async_copy(pfl.at[pl.ds(off * 16, _CC * 16)],
                             pbuf[b], sem[b])

        def wait(b, k):
            off = base + k * _CC
            pltpu.make_async_copy(ht.at[sadj[b]], hrows[b], sem[b]).wait()
            pltpu.make_async_copy(pfl.at[pl.ds(off * 16, _CC * 16)],
                                  pbuf[b], sem[b]).wait()

        def compute(b):
            def edge(j, cc):
                w = pbuf[b][pl.ds(j * 16, 16)]
                acc = [zero16] * 4
                for h in range(8):
                    whb = jnp.full((16,), w[h], jnp.float32)
                    for v in range(4):
                        hv = hrows[b][j, pl.ds(h * 64 + v * 16, 16)]
                        acc[v] = acc[v] + whb * hv
                for v in range(4):
                    msg[j, pl.ds(v * 16, 16)] = acc[v]
                return cc

            lax.fori_loop(0, _CC, edge, 0)
            pltpu.sync_copy(msg, sacc.at[didx[b]], add=True)

        for b in range(2):
            fetch(b, b)

        def outer(t, carry):
            for b in range(2):
                k = t * 2 + b
                wait(b, k)
                compute(b)
                fetch(b, k + 2)
            return carry

        lax.fori_loop(0, _NCHC // 2, outer, 0)
        for b in range(2):
            wait(b, _NCHC + b)
        plsc.subcore_barrier()

        def fcp(i, cc):
            off = s * stripe + i * 64
            fill_zidx(off)
            pltpu.sync_copy(sacc.at[zidx], msg)
            pltpu.sync_copy(msg, out.at[c, pl.ds(off, 64)])
            return cc

        lax.fori_loop(0, stripe // 64, fcp, 0)

    return functools.partial(
        pl.kernel,
        out_type=[jax.ShapeDtypeStruct((2, _NP, 64), jnp.float32)],
        mesh=_MESH,
        scratch_types=(
            [pltpu.VMEM((_CC,), jnp.int32)] * 6
            + [pltpu.VMEM((_CC, 512), jnp.float32)] * 2
            + [pltpu.VMEM((_CC * 16,), jnp.float32)] * 2
            + [
                pltpu.VMEM((_CC, 64), jnp.float32),
                pltpu.VMEM((64,), jnp.int32),
                pltpu.VMEM_SHARED((_NP, 64), jnp.float32),
                pltpu.SemaphoreType.DMA,
                pltpu.SemaphoreType.DMA,
            ]
        ),
    )(_msg_body)


_msg0 = _make_msg(0)
_msg1 = _make_msg(1)


_attn = functools.partial(
    pl.kernel,
    out_type=[
        jax.ShapeDtypeStruct((_EPP * 16,), jnp.float32),
        jax.ShapeDtypeStruct((2, _NP, 16), jnp.float32),
    ],
    mesh=_MESH,
    scratch_types=(
        [pltpu.VMEM((_CB,), jnp.int32)] * 4
        + [pltpu.VMEM((_CB, 128), jnp.float32)] * 4
        + [
            pltpu.VMEM((_CB, 16), jnp.float32),
            pltpu.VMEM((_CB * 16,), jnp.float32),
            pltpu.VMEM((_CB, 16), jnp.float32),
            pltpu.VMEM((_CB,), jnp.int32),
            pltpu.VMEM_SHARED((_NP, 16), jnp.float32),
            pltpu.SemaphoreType.DMA,
            pltpu.SemaphoreType.DMA,
        ]
    ),
)(_attn_body)


def _gat_conv(x, src, dst, srcp, dstp, z16, W, att_src, att_dst, bias):
    n = x.shape[0]
    heads, oc = att_src.shape
    k = W.shape[0]
    w_src = jnp.einsum('khc,hc->kh', W.reshape(k, heads, oc), att_src)
    w_dst = jnp.einsum('khc,hc->kh', W.reshape(k, heads, oc), att_dst)
    W_ext = jnp.concatenate(
        [W, w_src, w_dst, jnp.zeros((k, 112), jnp.float32)], axis=1)
    hx = _matmul(x, W_ext)
    h = hx[:, :heads * oc].reshape(n, heads, oc)
    a_src = hx[:, heads * oc:heads * oc + 8]
    a_dst = hx[:, heads * oc + 8:heads * oc + 16]
    asd = jnp.zeros((_NP, 128), jnp.float32)
    asd = asd.at[:n, :8].set(a_src).at[:n, 8:16].set(a_dst)
    dsa = jnp.zeros((_NP, 128), jnp.float32)
    dsa = dsa.at[:n, :8].set(a_dst).at[:n, 8:16].set(a_src)
    p_flat, s_part = _attn(asd, dsa, srcp, dstp, z16)
    s = s_part[0, :, :8] + s_part[1, :, :8]
    sinv = 1.0 / (8.0 * (s + 1e-16))
    stab = jnp.zeros((_NP, 128), jnp.float32).at[:, :8].set(sinv)
    ht = jnp.zeros((4 * _NP, 512), jnp.float32)
    for q in range(4):
        ht = ht.at[q * _NP:q * _NP + n].set(
            h[:, :, q * 64:(q + 1) * 64].reshape(n, 512))
    (w_flat,) = _norm(stab, p_flat, dstp)
    (outa,) = _msg0(ht, w_flat, srcp, dstp)
    (outb,) = _msg1(ht, w_flat, srcp, dstp)
    return jnp.concatenate(
        [outa[0, :n], outb[0, :n], outa[1, :n], outb[1, :n]], axis=1) + bias


def _pool_kernel(h2_ref, batch_ref, wl_ref, bl_ref, o_ref, acc, cnt):
    i = pl.program_id(0)

    @pl.when(i == 0)
    def _init():
        acc[...] = jnp.zeros_like(acc)
        cnt[...] = jnp.zeros_like(cnt)

    z = h2_ref[...]
    bt = batch_ref[0, 0, :]
    oh = (bt[:, None] == lax.broadcasted_iota(jnp.int32, (400, 64), 1)
          ).astype(jnp.float32)
    acc[...] += jnp.dot(oh.T, z, preferred_element_type=jnp.float32)
    cnt[...] += jnp.dot(oh.T, jnp.ones((400, 128), jnp.float32),
                        preferred_element_type=jnp.float32)

    @pl.when(i == 24)
    def _fin():
        pooled = acc[...] / jnp.maximum(cnt[...][:, 0:1], 1.0)
        o_ref[...] = (jnp.dot(pooled, wl_ref[...],
                              preferred_element_type=jnp.float32)
                      + bl_ref[...])


def _pool(h2, batch3, wl, bl2):
    return pl.pallas_call(
        _pool_kernel,
        grid=(25,),
        in_specs=[
            pl.BlockSpec((400, 256), lambda i: (i, 0)),
            pl.BlockSpec((1, 1, 400), lambda i: (i, 0, 0)),
            pl.BlockSpec((256, 64), lambda i: (0, 0)),
            pl.BlockSpec((1, 64), lambda i: (0, 0)),
        ],
        out_specs=pl.BlockSpec((64, 64), lambda i: (0, 0)),
        out_shape=jax.ShapeDtypeStruct((64, 64), jnp.float32),
        scratch_shapes=[
            pltpu.VMEM((64, 256), jnp.float32),
            pltpu.VMEM((64, 128), jnp.float32),
        ],
    )(h2, batch3, wl, bl2)


def kernel(x, edge_index, batch, W1, att_src1, att_dst1, b1,
           W2, att_src2, att_dst2, b2, Wl, bl):
    n = x.shape[0]
    loops = jnp.arange(n, dtype=edge_index.dtype)
    src = jnp.concatenate([edge_index[0], loops])
    dst = jnp.concatenate([edge_index[1], loops])
    pad = jnp.full((_EPP - _E_TOT,), _N, dtype=edge_index.dtype)
    srcp = jnp.concatenate([src, pad])
    dstp = jnp.concatenate([dst, pad])
    z16 = jnp.zeros((_NP, 16), jnp.float32)
    h = jax.nn.relu(
        _gat_conv(x, src, dst, srcp, dstp, z16, W1, att_src1, att_dst1, b1))
    h = _gat_conv(h, src, dst, srcp, dstp, z16, W2, att_src2, att_dst2, b2)
    batch3 = batch.reshape(25, 1, 400)
    return _pool(h, batch3, Wl, bl.reshape(1, 64))
